# Initial kernel scaffold; baseline (speedup 1.0000x reference)
import jax
import jax.numpy as jnp
from jax.experimental import pallas as pl


def kernel(momenta, types, atomic_masses):
    raise NotImplementedError("write your pallas kernel here")


# speedup vs baseline: 1.5683x; 1.5683x over previous
import functools

import jax
import jax.numpy as jnp
from jax import lax
from jax.experimental import pallas as pl
from jax.experimental.pallas import tpu as pltpu
from jax.experimental.pallas import tpu_sc as plsc

FS = 0.09822694788464063
SCALE = 0.25 * FS
TIMESTEPS = (1, 2, 4, 8)

_NC = 2
_NS = 16
_NW = _NC * _NS

_C = 2000
_UNROLL = 5


def _sc_gather_recip(types, masses_padded):
    n = types.shape[0]
    n_chunks = n // _C
    assert n % _C == 0
    n_per = -(-n_chunks // _NW)

    mesh = plsc.VectorSubcoreMesh(core_axis_name="c", subcore_axis_name="s")

    @functools.partial(
        pl.kernel,
        out_type=jax.ShapeDtypeStruct((n,), jnp.float32),
        mesh=mesh,
        compiler_params=pltpu.CompilerParams(needs_layout_passes=False),
        scratch_types=[
            pltpu.VMEM((112,), jnp.float32),
            pltpu.VMEM((112,), jnp.float32),
            pltpu.VMEM((_C,), jnp.int32),
            pltpu.VMEM((_C,), jnp.float32),
        ],
    )
    def sc_kernel(types_hbm, masses_hbm, out_hbm, tab_v, recip_v, idx_v, res_v):
        wid = lax.axis_index("s") * _NC + lax.axis_index("c")
        pltpu.sync_copy(masses_hbm, tab_v)
        for j in range(112 // 16):
            recip_v[pl.ds(j * 16, 16)] = SCALE / tab_v[pl.ds(j * 16, 16)]

        def chunk_body(k, _):
            ci = wid + k * _NW

            @pl.when(ci < n_chunks)
            def _():
                base = ci * _C
                pltpu.sync_copy(types_hbm.at[pl.ds(base, _C)], idx_v)

                def vec_body(i, _):
                    off = i * (16 * _UNROLL)
                    for j in range(_UNROLL):
                        o = off + j * 16
                        idx = idx_v[pl.ds(o, 16)]
                        res_v[pl.ds(o, 16)] = plsc.load_gather(recip_v, [idx])
                    return 0

                lax.fori_loop(0, _C // (16 * _UNROLL), vec_body, 0)
                pltpu.sync_copy(res_v, out_hbm.at[pl.ds(base, _C)])

            return 0

        lax.fori_loop(0, n_per, chunk_body, 0)

    return sc_kernel(types, masses_padded)


def _tc_scale(mom_v, rmass_v, sel):
    rows = mom_v.shape[0]
    br = 1000
    assert rows % br == 0
    grid = rows // br

    def body(sel_ref, mom_ref, rm_ref, out_ref):
        mb = jnp.dot(rm_ref[...], sel_ref[...], preferred_element_type=jnp.float32)
        base = mom_ref[...] * mb
        out_ref[0] = base
        base2 = base + base
        out_ref[1] = base2
        base4 = base2 + base2
        out_ref[2] = base4
        out_ref[3] = base4 + base4

    return pl.pallas_call(
        body,
        grid=(grid,),
        in_specs=[
            pl.BlockSpec((40, 120), lambda i: (0, 0)),
            pl.BlockSpec((br, 120), lambda i: (i, 0)),
            pl.BlockSpec((br, 40), lambda i: (i, 0)),
        ],
        out_specs=pl.BlockSpec((4, br, 120), lambda i: (0, i, 0)),
        out_shape=jax.ShapeDtypeStruct((4, rows, 120), jnp.float32),
    )(sel, mom_v, rmass_v)


def kernel(momenta, types, atomic_masses):
    n = types.shape[0]
    assert momenta.shape == (n, 3, 1)
    assert (3 * n) % 120 == 0
    rows = (3 * n) // 120

    masses_padded = jnp.concatenate(
        [atomic_masses, jnp.ones((112 - atomic_masses.shape[0],), jnp.float32)]
    )

    rmass = _sc_gather_recip(types, masses_padded)

    sel = jnp.repeat(jnp.eye(40, dtype=jnp.float32), 3, axis=1)

    mom_v = momenta.reshape(rows, 120)
    rmass_v = rmass.reshape(rows, 40)
    out = _tc_scale(mom_v, rmass_v, sel)

    delta_q = out.reshape(4, n, 3, 1)
    delta_p = jnp.zeros_like(delta_q)
    return (delta_p, delta_q)


# speedup vs baseline: 83.8277x; 53.4510x over previous
import functools

import jax
import jax.numpy as jnp
from jax import lax
from jax.experimental import pallas as pl
from jax.experimental.pallas import tpu as pltpu
from jax.experimental.pallas import tpu_sc as plsc

FS = 0.09822694788464063
SCALE = 0.25 * FS

_NC = 2
_NS = 16
_NW = _NC * _NS

_C = 2000
_UNROLL = 5

_BL = 128000


def _sc_gather_recip(types, masses_padded):
    n = types.shape[0]
    n_chunks = n // _C
    assert n % _C == 0
    n_per = -(-n_chunks // _NW)

    mesh = plsc.VectorSubcoreMesh(core_axis_name="c", subcore_axis_name="s")

    @functools.partial(
        pl.kernel,
        out_type=jax.ShapeDtypeStruct((n,), jnp.float32),
        mesh=mesh,
        compiler_params=pltpu.CompilerParams(needs_layout_passes=False),
        scratch_types=[
            pltpu.VMEM((112,), jnp.float32),
            pltpu.VMEM((112,), jnp.float32),
            pltpu.VMEM((_C,), jnp.int32),
            pltpu.VMEM((_C,), jnp.float32),
        ],
    )
    def sc_kernel(types_hbm, masses_hbm, out_hbm, tab_v, recip_v, idx_v, res_v):
        wid = lax.axis_index("s") * _NC + lax.axis_index("c")
        pltpu.sync_copy(masses_hbm, tab_v)
        for j in range(112 // 16):
            recip_v[pl.ds(j * 16, 16)] = SCALE / tab_v[pl.ds(j * 16, 16)]

        def chunk_body(k, _):
            ci = wid + k * _NW

            @pl.when(ci < n_chunks)
            def _():
                base = ci * _C
                pltpu.sync_copy(types_hbm.at[pl.ds(base, _C)], idx_v)

                def vec_body(i, _):
                    off = i * (16 * _UNROLL)
                    for j in range(_UNROLL):
                        o = off + j * 16
                        idx = idx_v[pl.ds(o, 16)]
                        res_v[pl.ds(o, 16)] = plsc.load_gather(recip_v, [idx])
                    return 0

                lax.fori_loop(0, _C // (16 * _UNROLL), vec_body, 0)
                pltpu.sync_copy(res_v, out_hbm.at[pl.ds(base, _C)])

            return 0

        lax.fori_loop(0, n_per, chunk_body, 0)

    return sc_kernel(types, masses_padded)


def _tc_scale(mom_t, rm):
    n = mom_t.shape[-1]
    grid = -(-n // _BL)

    def body(mom_ref, rm_ref, out_ref):
        m = mom_ref[:, 0, :]
        r = rm_ref[...].reshape(1, -1)
        base = m * r
        base2 = base + base
        base4 = base2 + base2
        out_ref[0, :, 0, :] = base
        out_ref[1, :, 0, :] = base2
        out_ref[2, :, 0, :] = base4
        out_ref[3, :, 0, :] = base4 + base4

    return pl.pallas_call(
        body,
        grid=(grid,),
        in_specs=[
            pl.BlockSpec((3, 1, _BL), lambda i: (0, 0, i)),
            pl.BlockSpec((_BL,), lambda i: (i,)),
        ],
        out_specs=pl.BlockSpec((4, 3, 1, _BL), lambda i: (0, 0, 0, i)),
        out_shape=jax.ShapeDtypeStruct((4, 3, 1, n), jnp.float32),
    )(mom_t, rm)


def kernel(momenta, types, atomic_masses):
    n = types.shape[0]
    assert momenta.shape == (n, 3, 1)

    masses_padded = jnp.concatenate(
        [atomic_masses, jnp.ones((112 - atomic_masses.shape[0],), jnp.float32)]
    )

    rm = _sc_gather_recip(types, masses_padded)
    mom_t = jnp.transpose(momenta, (1, 2, 0))
    out = _tc_scale(mom_t, rm)

    delta_q = jnp.transpose(out, (0, 3, 1, 2))
    delta_p = jnp.zeros_like(delta_q)
    return (delta_p, delta_q)


# speedup vs baseline: 98.7087x; 1.1775x over previous
import functools

import jax
import jax.numpy as jnp
from jax import lax
from jax.experimental import pallas as pl
from jax.experimental.pallas import tpu as pltpu
from jax.experimental.pallas import tpu_sc as plsc

FS = 0.09822694788464063
SCALE = 0.25 * FS

_NC = 2
_NS = 16
_NW = _NC * _NS

_C = 2000
_UNROLL = 5

_BL = 128000


def _sc_gather_recip(types, masses_padded):
    n = types.shape[0]
    n_chunks = n // _C
    assert n % _C == 0
    n_per = -(-n_chunks // _NW)

    mesh = plsc.VectorSubcoreMesh(core_axis_name="c", subcore_axis_name="s")

    @functools.partial(
        pl.kernel,
        out_type=jax.ShapeDtypeStruct((n,), jnp.float32),
        mesh=mesh,
        compiler_params=pltpu.CompilerParams(needs_layout_passes=False),
        scratch_types=[
            pltpu.VMEM((112,), jnp.float32),
            pltpu.VMEM((112,), jnp.float32),
            pltpu.VMEM((_C,), jnp.int32),
            pltpu.VMEM((_C,), jnp.float32),
        ],
    )
    def sc_kernel(types_hbm, masses_hbm, out_hbm, tab_v, recip_v, idx_v, res_v):
        wid = lax.axis_index("s") * _NC + lax.axis_index("c")
        pltpu.sync_copy(masses_hbm, tab_v)
        for j in range(112 // 16):
            recip_v[pl.ds(j * 16, 16)] = SCALE / tab_v[pl.ds(j * 16, 16)]

        def chunk_body(k, _):
            ci = wid + k * _NW

            @pl.when(ci < n_chunks)
            def _():
                base = ci * _C
                pltpu.sync_copy(types_hbm.at[pl.ds(base, _C)], idx_v)

                def vec_body(i, _):
                    off = i * (16 * _UNROLL)
                    for j in range(_UNROLL):
                        o = off + j * 16
                        idx = idx_v[pl.ds(o, 16)]
                        res_v[pl.ds(o, 16)] = plsc.load_gather(recip_v, [idx])
                    return 0

                lax.fori_loop(0, _C // (16 * _UNROLL), vec_body, 0)
                pltpu.sync_copy(res_v, out_hbm.at[pl.ds(base, _C)])

            return 0

        lax.fori_loop(0, n_per, chunk_body, 0)

    return sc_kernel(types, masses_padded)


def _tc_scale(mom_t, rm):
    n = mom_t.shape[-1]
    grid = -(-n // _BL)

    def body(mom_ref, rm_ref, out_ref, zero_ref):
        m = mom_ref[:, 0, :]
        r = rm_ref[...].reshape(1, -1)
        base = m * r
        base2 = base + base
        base4 = base2 + base2
        out_ref[0, :, 0, :] = base
        out_ref[1, :, 0, :] = base2
        out_ref[2, :, 0, :] = base4
        out_ref[3, :, 0, :] = base4 + base4
        zero_ref[...] = jnp.zeros(zero_ref.shape, jnp.float32)

    ospec = pl.BlockSpec((4, 3, 1, _BL), lambda i: (0, 0, 0, i))
    return pl.pallas_call(
        body,
        grid=(grid,),
        in_specs=[
            pl.BlockSpec((3, 1, _BL), lambda i: (0, 0, i)),
            pl.BlockSpec((_BL,), lambda i: (i,)),
        ],
        out_specs=[ospec, ospec],
        out_shape=[jax.ShapeDtypeStruct((4, 3, 1, n), jnp.float32)] * 2,
    )(mom_t, rm)


def kernel(momenta, types, atomic_masses):
    n = types.shape[0]
    assert momenta.shape == (n, 3, 1)

    masses_padded = jnp.concatenate(
        [atomic_masses, jnp.ones((112 - atomic_masses.shape[0],), jnp.float32)]
    )

    rm = _sc_gather_recip(types, masses_padded)
    mom_t = jnp.transpose(momenta, (1, 2, 0))
    out, out_p = _tc_scale(mom_t, rm)

    delta_q = jnp.transpose(out, (0, 3, 1, 2))
    delta_p = jnp.transpose(out_p, (0, 3, 1, 2))
    return (delta_p, delta_q)
